# baseline (speedup 1.0000x reference)
import functools

import jax
import jax.numpy as jnp
from jax import lax
from jax.experimental import pallas as pl
from jax.experimental.pallas import tpu as pltpu
from jax.experimental.pallas import tpu_sc as plsc

N_NODES = 10000
NP = 10240
N_EDGES = 320000
NC = 2
NS = 16
NW = NC * NS
CHUNK = 128
NCHUNKS = N_EDGES // CHUNK
RPT = NP // NS


def _sc_mesh():
    return plsc.VectorSubcoreMesh(
        core_axis_name="c", subcore_axis_name="s", num_cores=NC,
        num_subcores=NS)


def _ids():
    cid = lax.axis_index("c")
    sid = lax.axis_index("s")
    return cid * NS + sid, cid, sid


def _nchunks_for(wid):
    return (NCHUNKS - wid + NW - 1) // NW


def _sc_degrees(src, dst, zeros_np):
    @functools.partial(
        pl.kernel,
        out_type=tuple(
            jax.ShapeDtypeStruct((NP,), jnp.float32) for _ in range(4)),
        mesh=_sc_mesh(),
        scratch_types=[
            pltpu.VMEM((CHUNK,), jnp.int32),
            pltpu.VMEM((CHUNK,), jnp.int32),
            pltpu.VMEM((CHUNK,), jnp.float32),
            pltpu.VMEM_SHARED((NP,), jnp.float32),
            pltpu.VMEM_SHARED((NP,), jnp.float32),
        ],
    )
    def deg_kernel(src_hbm, dst_hbm, zeros_hbm,
                   o0_hbm, i0_hbm, o1_hbm, i1_hbm,
                   sidx, didx, ones_v, acc_out, acc_in):
        wid, cid, sid = _ids()

        for k in range(CHUNK // 16):
            ones_v[pl.ds(k * 16, 16)] = jnp.ones((16,), jnp.float32)

        r0 = sid * RPT
        pltpu.sync_copy(zeros_hbm.at[pl.ds(r0, RPT)], acc_out.at[pl.ds(r0, RPT)])
        pltpu.sync_copy(zeros_hbm.at[pl.ds(r0, RPT)], acc_in.at[pl.ds(r0, RPT)])
        plsc.subcore_barrier()

        def body(i, carry):
            base = (wid + NW * i) * CHUNK
            pltpu.sync_copy(src_hbm.at[pl.ds(base, CHUNK)], sidx)
            pltpu.sync_copy(dst_hbm.at[pl.ds(base, CHUNK)], didx)
            pltpu.sync_copy(ones_v, acc_out.at[sidx], add=True)
            pltpu.sync_copy(ones_v, acc_in.at[didx], add=True)
            return carry

        lax.fori_loop(0, _nchunks_for(wid), body, 0)
        plsc.subcore_barrier()

        @pl.when(cid == 0)
        def _():
            pltpu.sync_copy(acc_out.at[pl.ds(r0, RPT)], o0_hbm.at[pl.ds(r0, RPT)])
            pltpu.sync_copy(acc_in.at[pl.ds(r0, RPT)], i0_hbm.at[pl.ds(r0, RPT)])

        @pl.when(cid == 1)
        def _():
            pltpu.sync_copy(acc_out.at[pl.ds(r0, RPT)], o1_hbm.at[pl.ds(r0, RPT)])
            pltpu.sync_copy(acc_in.at[pl.ds(r0, RPT)], i1_hbm.at[pl.ds(r0, RPT)])

    return deg_kernel(src, dst, zeros_np)


NLANE = 4
ECHUNK = 80
ECH = 4096
PC = ECH // NW
TRASH = NP
NPA = NP + 8


def _sc_gather_scatter(hs, idx3, zeros_npf, feat):
    @functools.partial(
        pl.kernel,
        out_type=tuple(
            jax.ShapeDtypeStruct((NP, feat), jnp.float32) for _ in range(2)),
        mesh=_sc_mesh(),
        scratch_types=(
            [pltpu.VMEM((2, ECHUNK), jnp.int32) for _ in range(NLANE)]
            + [pltpu.VMEM((ECHUNK, feat), jnp.float32) for _ in range(NLANE)]
            + [pltpu.VMEM_SHARED((NPA, feat), jnp.float32)]
            + [pltpu.SemaphoreType.DMA for _ in range(NLANE)]
        ),
    )
    def edge_kernel(hs_hbm, idx3_hbm, zeros_hbm,
                    out0_hbm, out1_hbm, *refs):
        idxs = refs[:NLANE]
        rows = refs[NLANE:2 * NLANE]
        acc = refs[2 * NLANE]
        sems = refs[2 * NLANE + 1:2 * NLANE + 1 + NLANE]
        wid, cid, sid = _ids()

        r0 = sid * RPT
        pltpu.sync_copy(zeros_hbm.at[pl.ds(r0, RPT)], acc.at[pl.ds(r0, RPT)])
        plsc.subcore_barrier()

        def body(i, carry):
            cps = []
            for j in range(NLANE):
                pltpu.sync_copy(
                    idx3_hbm.at[wid + NW * (NLANE * i + j)], idxs[j])
                cps.append(pltpu.async_copy(
                    hs_hbm.at[idxs[j].at[0]], rows[j], sems[j]))
            for j in range(NLANE):
                cps[j].wait()
                pltpu.sync_copy(rows[j], acc.at[idxs[j].at[1]], add=True)
            return carry

        lax.fori_loop(0, PC // NLANE, body, 0)
        plsc.subcore_barrier()

        @pl.when(cid == 0)
        def _():
            pltpu.sync_copy(acc.at[pl.ds(r0, RPT)], out0_hbm.at[pl.ds(r0, RPT)])

        @pl.when(cid == 1)
        def _():
            pltpu.sync_copy(acc.at[pl.ds(r0, RPT)], out1_hbm.at[pl.ds(r0, RPT)])

    return edge_kernel(hs, idx3, zeros_npf)


_BM = 1000


def _tc_matmul_srcnorm(x, w, degp_out):
    fin = x.shape[1]
    fout = w.shape[1]

    def body(x_ref, w_ref, dp_ref, o_ref):
        deg = dp_ref[:, 0] + dp_ref[:, 1]
        norm = jnp.where(deg > 0, lax.rsqrt(deg), 0.0)
        acc = jnp.dot(x_ref[...], w_ref[...],
                      preferred_element_type=jnp.float32)
        o_ref[...] = acc * norm[:, None]

    return pl.pallas_call(
        body,
        grid=(N_NODES // _BM,),
        in_specs=[
            pl.BlockSpec((_BM, fin), lambda i: (i, 0)),
            pl.BlockSpec((fin, fout), lambda i: (0, 0)),
            pl.BlockSpec((_BM, NC), lambda i: (i, 0)),
        ],
        out_specs=pl.BlockSpec((_BM, fout), lambda i: (i, 0)),
        out_shape=jax.ShapeDtypeStruct((N_NODES, fout), jnp.float32),
    )(x, w, degp_out)


def _tc_combine(agg0, agg1, degp_in, b):
    feat = agg0.shape[1]

    def body(a0_ref, a1_ref, dp_ref, b_ref, o_ref):
        agg = a0_ref[...] + a1_ref[...]
        deg = dp_ref[:, 0] + dp_ref[:, 1]
        norm = jnp.where(deg > 0, lax.rsqrt(deg), 0.0)
        o_ref[...] = agg * norm[:, None] + b_ref[...][None, :]

    return pl.pallas_call(
        body,
        grid=(N_NODES // _BM,),
        in_specs=[
            pl.BlockSpec((_BM, feat), lambda i: (i, 0)),
            pl.BlockSpec((_BM, feat), lambda i: (i, 0)),
            pl.BlockSpec((_BM, NC), lambda i: (i, 0)),
            pl.BlockSpec((feat,), lambda i: (0,)),
        ],
        out_specs=pl.BlockSpec((_BM, feat), lambda i: (i, 0)),
        out_shape=jax.ShapeDtypeStruct((N_NODES, feat), jnp.float32),
    )(agg0, agg1, degp_in, b)


def kernel(features, edge_index, W1, b1, W2, b2, W3, b3):
    src = edge_index[0].astype(jnp.int32)
    dst = edge_index[1].astype(jnp.int32)

    zeros_np = jnp.zeros((NP,), jnp.float32)
    o0, i0, o1, i1 = _sc_degrees(src, dst, zeros_np)
    degp_out = jnp.stack([o0[:N_NODES], o1[:N_NODES]], axis=1)
    degp_in = jnp.stack([i0[:N_NODES], i1[:N_NODES]], axis=1)

    nfake = ECH * ECHUNK - N_EDGES
    srcp = jnp.concatenate([src, jnp.zeros((nfake,), jnp.int32)])
    dstp = jnp.concatenate([dst, jnp.full((nfake,), TRASH, jnp.int32)])
    idx3 = jnp.stack([srcp.reshape(ECH, ECHUNK), dstp.reshape(ECH, ECHUNK)],
                     axis=1)

    w3p = jnp.zeros((W3.shape[0], 128), jnp.float32).at[:, :40].set(W3)
    b3p = jnp.zeros((128,), jnp.float32).at[:40].set(b3)

    zeros128 = jnp.zeros((NP, 128), jnp.float32)

    h = features
    for w, b, zf in ((W1, b1, zeros128),
                     (W2, b2, zeros128),
                     (w3p, b3p, zeros128)):
        hs = _tc_matmul_srcnorm(h, w, degp_out)
        a0, a1 = _sc_gather_scatter(hs, idx3, zf, w.shape[1])
        h = _tc_combine(a0[:N_NODES], a1[:N_NODES], degp_in, b)

    return h[:, :40]


# speedup vs baseline: 1.6597x; 1.6597x over previous
import functools

import jax
import jax.numpy as jnp
from jax import lax
from jax.experimental import pallas as pl
from jax.experimental.pallas import tpu as pltpu
from jax.experimental.pallas import tpu_sc as plsc

N_NODES = 10000
NP = 10240
N_EDGES = 320000
NC = 2
NS = 16
NW = NC * NS
CHUNK = 128
NCHUNKS = N_EDGES // CHUNK
RPT = NP // NS


def _sc_mesh():
    return plsc.VectorSubcoreMesh(
        core_axis_name="c", subcore_axis_name="s", num_cores=NC,
        num_subcores=NS)


def _ids():
    cid = lax.axis_index("c")
    sid = lax.axis_index("s")
    return cid * NS + sid, cid, sid


def _nchunks_for(wid):
    return (NCHUNKS - wid + NW - 1) // NW


def _sc_degrees(src, dst, zeros_np):
    @functools.partial(
        pl.kernel,
        out_type=tuple(
            jax.ShapeDtypeStruct((NP,), jnp.float32) for _ in range(4)),
        mesh=_sc_mesh(),
        scratch_types=[
            pltpu.VMEM((CHUNK,), jnp.int32),
            pltpu.VMEM((CHUNK,), jnp.int32),
            pltpu.VMEM((CHUNK,), jnp.float32),
            pltpu.VMEM_SHARED((NP,), jnp.float32),
            pltpu.VMEM_SHARED((NP,), jnp.float32),
        ],
    )
    def deg_kernel(src_hbm, dst_hbm, zeros_hbm,
                   o0_hbm, i0_hbm, o1_hbm, i1_hbm,
                   sidx, didx, ones_v, acc_out, acc_in):
        wid, cid, sid = _ids()

        for k in range(CHUNK // 16):
            ones_v[pl.ds(k * 16, 16)] = jnp.ones((16,), jnp.float32)

        r0 = sid * RPT
        pltpu.sync_copy(zeros_hbm.at[pl.ds(r0, RPT)], acc_out.at[pl.ds(r0, RPT)])
        pltpu.sync_copy(zeros_hbm.at[pl.ds(r0, RPT)], acc_in.at[pl.ds(r0, RPT)])
        plsc.subcore_barrier()

        def body(i, carry):
            base = (wid + NW * i) * CHUNK
            pltpu.sync_copy(src_hbm.at[pl.ds(base, CHUNK)], sidx)
            pltpu.sync_copy(dst_hbm.at[pl.ds(base, CHUNK)], didx)
            pltpu.sync_copy(ones_v, acc_out.at[sidx], add=True)
            pltpu.sync_copy(ones_v, acc_in.at[didx], add=True)
            return carry

        lax.fori_loop(0, _nchunks_for(wid), body, 0)
        plsc.subcore_barrier()

        @pl.when(cid == 0)
        def _():
            pltpu.sync_copy(acc_out.at[pl.ds(r0, RPT)], o0_hbm.at[pl.ds(r0, RPT)])
            pltpu.sync_copy(acc_in.at[pl.ds(r0, RPT)], i0_hbm.at[pl.ds(r0, RPT)])

        @pl.when(cid == 1)
        def _():
            pltpu.sync_copy(acc_out.at[pl.ds(r0, RPT)], o1_hbm.at[pl.ds(r0, RPT)])
            pltpu.sync_copy(acc_in.at[pl.ds(r0, RPT)], i1_hbm.at[pl.ds(r0, RPT)])

    return deg_kernel(src, dst, zeros_np)


NLANE = 3
ECHUNK = 112
ECH = 2880
PC = ECH // NW
TRASH = NP
NPA = NP + 8


def _sc_gather_scatter(hs, idx3, zeros_npf, feat):
    @functools.partial(
        pl.kernel,
        out_type=tuple(
            jax.ShapeDtypeStruct((NP, feat), jnp.float32) for _ in range(2)),
        mesh=_sc_mesh(),
        scratch_types=(
            [pltpu.VMEM((2, ECHUNK), jnp.int32) for _ in range(NLANE)]
            + [pltpu.VMEM((ECHUNK, feat), jnp.float32) for _ in range(NLANE)]
            + [pltpu.VMEM_SHARED((NPA, feat), jnp.float32)]
            + [pltpu.SemaphoreType.DMA for _ in range(NLANE)]
        ),
    )
    def edge_kernel(hs_hbm, idx3_hbm, zeros_hbm,
                    out0_hbm, out1_hbm, *refs):
        idxs = refs[:NLANE]
        rows = refs[NLANE:2 * NLANE]
        acc = refs[2 * NLANE]
        sems = refs[2 * NLANE + 1:2 * NLANE + 1 + NLANE]
        wid, cid, sid = _ids()

        r0 = sid * RPT
        pltpu.sync_copy(zeros_hbm.at[pl.ds(r0, RPT)], acc.at[pl.ds(r0, RPT)])
        plsc.subcore_barrier()

        def body(i, carry):
            cps = []
            for j in range(NLANE):
                pltpu.sync_copy(
                    idx3_hbm.at[wid + NW * (NLANE * i + j)], idxs[j])
                cps.append(pltpu.async_copy(
                    hs_hbm.at[idxs[j].at[0]], rows[j], sems[j]))
            for j in range(NLANE):
                cps[j].wait()
                pltpu.sync_copy(rows[j], acc.at[idxs[j].at[1]], add=True)
            return carry

        lax.fori_loop(0, PC // NLANE, body, 0)
        plsc.subcore_barrier()

        @pl.when(cid == 0)
        def _():
            pltpu.sync_copy(acc.at[pl.ds(r0, RPT)], out0_hbm.at[pl.ds(r0, RPT)])

        @pl.when(cid == 1)
        def _():
            pltpu.sync_copy(acc.at[pl.ds(r0, RPT)], out1_hbm.at[pl.ds(r0, RPT)])

    return edge_kernel(hs, idx3, zeros_npf)


_BM = 1000


def _tc_matmul_srcnorm(x, w, degp_out):
    fin = x.shape[1]
    fout = w.shape[1]

    def body(x_ref, w_ref, dp_ref, o_ref):
        deg = dp_ref[:, 0] + dp_ref[:, 1]
        norm = jnp.where(deg > 0, lax.rsqrt(deg), 0.0)
        acc = jnp.dot(x_ref[...], w_ref[...],
                      preferred_element_type=jnp.float32)
        o_ref[...] = acc * norm[:, None]

    return pl.pallas_call(
        body,
        grid=(N_NODES // _BM,),
        in_specs=[
            pl.BlockSpec((_BM, fin), lambda i: (i, 0)),
            pl.BlockSpec((fin, fout), lambda i: (0, 0)),
            pl.BlockSpec((_BM, NC), lambda i: (i, 0)),
        ],
        out_specs=pl.BlockSpec((_BM, fout), lambda i: (i, 0)),
        out_shape=jax.ShapeDtypeStruct((N_NODES, fout), jnp.float32),
    )(x, w, degp_out)


def _tc_combine(agg0, agg1, degp_in, b):
    feat = agg0.shape[1]

    def body(a0_ref, a1_ref, dp_ref, b_ref, o_ref):
        agg = a0_ref[...] + a1_ref[...]
        deg = dp_ref[:, 0] + dp_ref[:, 1]
        norm = jnp.where(deg > 0, lax.rsqrt(deg), 0.0)
        o_ref[...] = agg * norm[:, None] + b_ref[...][None, :]

    return pl.pallas_call(
        body,
        grid=(N_NODES // _BM,),
        in_specs=[
            pl.BlockSpec((_BM, feat), lambda i: (i, 0)),
            pl.BlockSpec((_BM, feat), lambda i: (i, 0)),
            pl.BlockSpec((_BM, NC), lambda i: (i, 0)),
            pl.BlockSpec((feat,), lambda i: (0,)),
        ],
        out_specs=pl.BlockSpec((_BM, feat), lambda i: (i, 0)),
        out_shape=jax.ShapeDtypeStruct((N_NODES, feat), jnp.float32),
    )(agg0, agg1, degp_in, b)


def kernel(features, edge_index, W1, b1, W2, b2, W3, b3):
    src = edge_index[0].astype(jnp.int32)
    dst = edge_index[1].astype(jnp.int32)

    zeros_np = jnp.zeros((NP,), jnp.float32)
    o0, i0, o1, i1 = _sc_degrees(src, dst, zeros_np)
    degp_out = jnp.stack([o0[:N_NODES], o1[:N_NODES]], axis=1)
    degp_in = jnp.stack([i0[:N_NODES], i1[:N_NODES]], axis=1)

    nfake = ECH * ECHUNK - N_EDGES
    srcp = jnp.concatenate([src, jnp.zeros((nfake,), jnp.int32)])
    dstp = jnp.concatenate([dst, jnp.full((nfake,), TRASH, jnp.int32)])
    idx3 = jnp.stack([srcp.reshape(ECH, ECHUNK), dstp.reshape(ECH, ECHUNK)],
                     axis=1)

    w3p = jnp.zeros((W3.shape[0], 128), jnp.float32).at[:, :40].set(W3)
    b3p = jnp.zeros((128,), jnp.float32).at[:40].set(b3)

    zeros128 = jnp.zeros((NP, 128), jnp.float32)

    h = features
    for w, b, zf in ((W1, b1, zeros128),
                     (W2, b2, zeros128),
                     (w3p, b3p, zeros128)):
        hs = _tc_matmul_srcnorm(h, w, degp_out)
        a0, a1 = _sc_gather_scatter(hs, idx3, zf, w.shape[1])
        h = _tc_combine(a0[:N_NODES], a1[:N_NODES], degp_in, b)

    return h[:, :40]


# speedup vs baseline: 1.6644x; 1.0028x over previous
import functools

import jax
import jax.numpy as jnp
from jax import lax
from jax.experimental import pallas as pl
from jax.experimental.pallas import tpu as pltpu
from jax.experimental.pallas import tpu_sc as plsc

N_NODES = 10000
NP = 10240
N_EDGES = 320000
NC = 2
NS = 16
NW = NC * NS
CHUNK = 128
NCHUNKS = N_EDGES // CHUNK
RPT = NP // NS


def _sc_mesh():
    return plsc.VectorSubcoreMesh(
        core_axis_name="c", subcore_axis_name="s", num_cores=NC,
        num_subcores=NS)


def _ids():
    cid = lax.axis_index("c")
    sid = lax.axis_index("s")
    return cid * NS + sid, cid, sid


def _nchunks_for(wid):
    return (NCHUNKS - wid + NW - 1) // NW


def _sc_degrees(src, dst, zeros_np):
    @functools.partial(
        pl.kernel,
        out_type=tuple(
            jax.ShapeDtypeStruct((NP,), jnp.float32) for _ in range(4)),
        mesh=_sc_mesh(),
        scratch_types=[
            pltpu.VMEM((CHUNK,), jnp.int32),
            pltpu.VMEM((CHUNK,), jnp.int32),
            pltpu.VMEM((CHUNK,), jnp.float32),
            pltpu.VMEM_SHARED((NP,), jnp.float32),
            pltpu.VMEM_SHARED((NP,), jnp.float32),
        ],
    )
    def deg_kernel(src_hbm, dst_hbm, zeros_hbm,
                   o0_hbm, i0_hbm, o1_hbm, i1_hbm,
                   sidx, didx, ones_v, acc_out, acc_in):
        wid, cid, sid = _ids()

        for k in range(CHUNK // 16):
            ones_v[pl.ds(k * 16, 16)] = jnp.ones((16,), jnp.float32)

        r0 = sid * RPT
        pltpu.sync_copy(zeros_hbm.at[pl.ds(r0, RPT)], acc_out.at[pl.ds(r0, RPT)])
        pltpu.sync_copy(zeros_hbm.at[pl.ds(r0, RPT)], acc_in.at[pl.ds(r0, RPT)])
        plsc.subcore_barrier()

        def body(i, carry):
            base = (wid + NW * i) * CHUNK
            pltpu.sync_copy(src_hbm.at[pl.ds(base, CHUNK)], sidx)
            pltpu.sync_copy(dst_hbm.at[pl.ds(base, CHUNK)], didx)
            pltpu.sync_copy(ones_v, acc_out.at[sidx], add=True)
            pltpu.sync_copy(ones_v, acc_in.at[didx], add=True)
            return carry

        lax.fori_loop(0, _nchunks_for(wid), body, 0)
        plsc.subcore_barrier()

        @pl.when(cid == 0)
        def _():
            pltpu.sync_copy(acc_out.at[pl.ds(r0, RPT)], o0_hbm.at[pl.ds(r0, RPT)])
            pltpu.sync_copy(acc_in.at[pl.ds(r0, RPT)], i0_hbm.at[pl.ds(r0, RPT)])

        @pl.when(cid == 1)
        def _():
            pltpu.sync_copy(acc_out.at[pl.ds(r0, RPT)], o1_hbm.at[pl.ds(r0, RPT)])
            pltpu.sync_copy(acc_in.at[pl.ds(r0, RPT)], i1_hbm.at[pl.ds(r0, RPT)])

    return deg_kernel(src, dst, zeros_np)


NLANE = 3
ECHUNK = 120
ECH = 2688
PC = ECH // NW
TRASH = NP
NPA = NP + 8


def _sc_gather_scatter(hs, idx3, zeros_npf, feat):
    @functools.partial(
        pl.kernel,
        out_type=tuple(
            jax.ShapeDtypeStruct((NP, feat), jnp.float32) for _ in range(2)),
        mesh=_sc_mesh(),
        scratch_types=(
            [pltpu.VMEM((2, ECHUNK), jnp.int32) for _ in range(NLANE)]
            + [pltpu.VMEM((ECHUNK, feat), jnp.float32) for _ in range(NLANE)]
            + [pltpu.VMEM_SHARED((NPA, feat), jnp.float32)]
            + [pltpu.SemaphoreType.DMA for _ in range(NLANE)]
        ),
    )
    def edge_kernel(hs_hbm, idx3_hbm, zeros_hbm,
                    out0_hbm, out1_hbm, *refs):
        idxs = refs[:NLANE]
        rows = refs[NLANE:2 * NLANE]
        acc = refs[2 * NLANE]
        sems = refs[2 * NLANE + 1:2 * NLANE + 1 + NLANE]
        wid, cid, sid = _ids()

        r0 = sid * RPT
        pltpu.sync_copy(zeros_hbm.at[pl.ds(r0, RPT)], acc.at[pl.ds(r0, RPT)])
        plsc.subcore_barrier()

        def body(i, carry):
            cps = []
            for j in range(NLANE):
                pltpu.sync_copy(
                    idx3_hbm.at[wid + NW * (NLANE * i + j)], idxs[j])
                cps.append(pltpu.async_copy(
                    hs_hbm.at[idxs[j].at[0]], rows[j], sems[j]))
            for j in range(NLANE):
                cps[j].wait()
                pltpu.sync_copy(rows[j], acc.at[idxs[j].at[1]], add=True)
            return carry

        lax.fori_loop(0, PC // NLANE, body, 0)
        plsc.subcore_barrier()

        @pl.when(cid == 0)
        def _():
            pltpu.sync_copy(acc.at[pl.ds(r0, RPT)], out0_hbm.at[pl.ds(r0, RPT)])

        @pl.when(cid == 1)
        def _():
            pltpu.sync_copy(acc.at[pl.ds(r0, RPT)], out1_hbm.at[pl.ds(r0, RPT)])

    return edge_kernel(hs, idx3, zeros_npf)


_BM = 1000


def _tc_matmul_srcnorm(x, w, degp_out):
    fin = x.shape[1]
    fout = w.shape[1]

    def body(x_ref, w_ref, dp_ref, o_ref):
        deg = dp_ref[:, 0] + dp_ref[:, 1]
        norm = jnp.where(deg > 0, lax.rsqrt(deg), 0.0)
        acc = jnp.dot(x_ref[...], w_ref[...],
                      preferred_element_type=jnp.float32)
        o_ref[...] = acc * norm[:, None]

    return pl.pallas_call(
        body,
        grid=(N_NODES // _BM,),
        in_specs=[
            pl.BlockSpec((_BM, fin), lambda i: (i, 0)),
            pl.BlockSpec((fin, fout), lambda i: (0, 0)),
            pl.BlockSpec((_BM, NC), lambda i: (i, 0)),
        ],
        out_specs=pl.BlockSpec((_BM, fout), lambda i: (i, 0)),
        out_shape=jax.ShapeDtypeStruct((N_NODES, fout), jnp.float32),
    )(x, w, degp_out)


def _tc_combine(agg0, agg1, degp_in, b):
    feat = agg0.shape[1]

    def body(a0_ref, a1_ref, dp_ref, b_ref, o_ref):
        agg = a0_ref[...] + a1_ref[...]
        deg = dp_ref[:, 0] + dp_ref[:, 1]
        norm = jnp.where(deg > 0, lax.rsqrt(deg), 0.0)
        o_ref[...] = agg * norm[:, None] + b_ref[...][None, :]

    return pl.pallas_call(
        body,
        grid=(N_NODES // _BM,),
        in_specs=[
            pl.BlockSpec((_BM, feat), lambda i: (i, 0)),
            pl.BlockSpec((_BM, feat), lambda i: (i, 0)),
            pl.BlockSpec((_BM, NC), lambda i: (i, 0)),
            pl.BlockSpec((feat,), lambda i: (0,)),
        ],
        out_specs=pl.BlockSpec((_BM, feat), lambda i: (i, 0)),
        out_shape=jax.ShapeDtypeStruct((N_NODES, feat), jnp.float32),
    )(agg0, agg1, degp_in, b)


def kernel(features, edge_index, W1, b1, W2, b2, W3, b3):
    src = edge_index[0].astype(jnp.int32)
    dst = edge_index[1].astype(jnp.int32)

    zeros_np = jnp.zeros((NP,), jnp.float32)
    o0, i0, o1, i1 = _sc_degrees(src, dst, zeros_np)
    degp_out = jnp.stack([o0[:N_NODES], o1[:N_NODES]], axis=1)
    degp_in = jnp.stack([i0[:N_NODES], i1[:N_NODES]], axis=1)

    nfake = ECH * ECHUNK - N_EDGES
    srcp = jnp.concatenate([src, jnp.zeros((nfake,), jnp.int32)])
    dstp = jnp.concatenate([dst, jnp.full((nfake,), TRASH, jnp.int32)])
    idx3 = jnp.stack([srcp.reshape(ECH, ECHUNK), dstp.reshape(ECH, ECHUNK)],
                     axis=1)

    w3p = jnp.zeros((W3.shape[0], 128), jnp.float32).at[:, :40].set(W3)
    b3p = jnp.zeros((128,), jnp.float32).at[:40].set(b3)

    zeros128 = jnp.zeros((NP, 128), jnp.float32)

    h = features
    for w, b, zf in ((W1, b1, zeros128),
                     (W2, b2, zeros128),
                     (w3p, b3p, zeros128)):
        hs = _tc_matmul_srcnorm(h, w, degp_out)
        a0, a1 = _sc_gather_scatter(hs, idx3, zf, w.shape[1])
        h = _tc_combine(a0[:N_NODES], a1[:N_NODES], degp_in, b)

    return h[:, :40]
